# baseline (speedup 1.0000x reference)
import jax
import jax.numpy as jnp
from jax.experimental import pallas as pl

_BR = 400


def _xw_body(x_ref, w_ref, o_ref):
    o_ref[...] = jnp.dot(x_ref[...], w_ref[...],
                         preferred_element_type=jnp.float32)


def _l1_body(adj_ref, s_ref, b_ref, w2_ref, w3_ref, o_ref):
    h = jnp.dot(adj_ref[...], s_ref[...], preferred_element_type=jnp.float32)
    h = jnp.maximum(h + b_ref[...], 0.0)
    hw2 = jnp.dot(h, w2_ref[...], preferred_element_type=jnp.float32)
    o_ref[...] = jnp.dot(hw2, w3_ref[...], preferred_element_type=jnp.float32)


def _l2_body(adj_ref, s_ref, b2_ref, w3_ref, o_ref):
    c = jnp.dot(b2_ref[...], w3_ref[...], preferred_element_type=jnp.float32)
    o_ref[...] = jnp.dot(adj_ref[...], s_ref[...],
                         preferred_element_type=jnp.float32) + c


def _l3_body(adj_ref, s_ref, b_ref, o_ref):
    h = jnp.dot(adj_ref[...], s_ref[...],
                preferred_element_type=jnp.float32) + b_ref[...]
    m = jnp.max(h, axis=1, keepdims=True)
    lse = jnp.log(jnp.sum(jnp.exp(h - m), axis=1, keepdims=True))
    o_ref[...] = h - m - lse


def kernel(x, adj, W1, b1, W2, b2, W3, b3):
    N, F = x.shape
    d1 = W1.shape[1]
    d2 = W2.shape[1]
    d3 = W3.shape[1]
    b1r = b1.reshape(1, d1)
    b2r = b2.reshape(1, d2)
    b3r = b3.reshape(1, d3)

    grid = (N // _BR,)
    row = lambda i: (i, 0)
    const = lambda i: (0, 0)

    s1 = pl.pallas_call(
        _xw_body,
        grid=grid,
        in_specs=[pl.BlockSpec((_BR, F), row),
                  pl.BlockSpec((F, d1), const)],
        out_specs=pl.BlockSpec((_BR, d1), row),
        out_shape=jax.ShapeDtypeStruct((N, d1), jnp.float32),
    )(x, W1)

    u = pl.pallas_call(
        _l1_body,
        grid=grid,
        in_specs=[pl.BlockSpec((_BR, N), row),
                  pl.BlockSpec((N, d1), const),
                  pl.BlockSpec((1, d1), const),
                  pl.BlockSpec((d1, d2), const),
                  pl.BlockSpec((d2, d3), const)],
        out_specs=pl.BlockSpec((_BR, d3), row),
        out_shape=jax.ShapeDtypeStruct((N, d3), jnp.float32),
    )(adj, s1, b1r, W2, W3)

    t = pl.pallas_call(
        _l2_body,
        grid=grid,
        in_specs=[pl.BlockSpec((_BR, N), row),
                  pl.BlockSpec((N, d3), const),
                  pl.BlockSpec((1, d2), const),
                  pl.BlockSpec((d2, d3), const)],
        out_specs=pl.BlockSpec((_BR, d3), row),
        out_shape=jax.ShapeDtypeStruct((N, d3), jnp.float32),
    )(adj, u, b2r, W3)

    out = pl.pallas_call(
        _l3_body,
        grid=grid,
        in_specs=[pl.BlockSpec((_BR, N), row),
                  pl.BlockSpec((N, d3), const),
                  pl.BlockSpec((1, d3), const)],
        out_specs=pl.BlockSpec((_BR, d3), row),
        out_shape=jax.ShapeDtypeStruct((N, d3), jnp.float32),
    )(adj, t, b3r)
    return out


# speedup vs baseline: 1.0086x; 1.0086x over previous
import jax
import jax.numpy as jnp
from jax.experimental import pallas as pl

_BR = 400


def _xw_body(x_ref, w_ref, o_ref):
    o_ref[...] = jnp.dot(x_ref[...].astype(jnp.bfloat16),
                         w_ref[...].astype(jnp.bfloat16),
                         preferred_element_type=jnp.float32)


def _l1_body(adj_ref, s_ref, b_ref, w2_ref, w3_ref, o_ref):
    h = jnp.dot(adj_ref[...].astype(jnp.bfloat16),
                s_ref[...].astype(jnp.bfloat16),
                preferred_element_type=jnp.float32)
    h = jnp.maximum(h + b_ref[...], 0.0)
    hw2 = jnp.dot(h, w2_ref[...], preferred_element_type=jnp.float32)
    o_ref[...] = jnp.dot(hw2, w3_ref[...], preferred_element_type=jnp.float32)


def _l2_body(adj_ref, s_ref, b2_ref, w3_ref, o_ref):
    c = jnp.dot(b2_ref[...], w3_ref[...], preferred_element_type=jnp.float32)
    o_ref[...] = jnp.dot(adj_ref[...].astype(jnp.bfloat16),
                         s_ref[...].astype(jnp.bfloat16),
                         preferred_element_type=jnp.float32) + c


def _l3_body(adj_ref, s_ref, b_ref, o_ref):
    h = jnp.dot(adj_ref[...].astype(jnp.bfloat16),
                s_ref[...].astype(jnp.bfloat16),
                preferred_element_type=jnp.float32) + b_ref[...]
    m = jnp.max(h, axis=1, keepdims=True)
    lse = jnp.log(jnp.sum(jnp.exp(h - m), axis=1, keepdims=True))
    o_ref[...] = h - m - lse


def kernel(x, adj, W1, b1, W2, b2, W3, b3):
    N, F = x.shape
    d1 = W1.shape[1]
    d2 = W2.shape[1]
    d3 = W3.shape[1]
    b1r = b1.reshape(1, d1)
    b2r = b2.reshape(1, d2)
    b3r = b3.reshape(1, d3)

    grid = (N // _BR,)
    row = lambda i: (i, 0)
    const = lambda i: (0, 0)

    s1 = pl.pallas_call(
        _xw_body,
        grid=grid,
        in_specs=[pl.BlockSpec((_BR, F), row),
                  pl.BlockSpec((F, d1), const)],
        out_specs=pl.BlockSpec((_BR, d1), row),
        out_shape=jax.ShapeDtypeStruct((N, d1), jnp.float32),
    )(x, W1)

    u = pl.pallas_call(
        _l1_body,
        grid=grid,
        in_specs=[pl.BlockSpec((_BR, N), row),
                  pl.BlockSpec((N, d1), const),
                  pl.BlockSpec((1, d1), const),
                  pl.BlockSpec((d1, d2), const),
                  pl.BlockSpec((d2, d3), const)],
        out_specs=pl.BlockSpec((_BR, d3), row),
        out_shape=jax.ShapeDtypeStruct((N, d3), jnp.float32),
    )(adj, s1, b1r, W2, W3)

    t = pl.pallas_call(
        _l2_body,
        grid=grid,
        in_specs=[pl.BlockSpec((_BR, N), row),
                  pl.BlockSpec((N, d3), const),
                  pl.BlockSpec((1, d2), const),
                  pl.BlockSpec((d2, d3), const)],
        out_specs=pl.BlockSpec((_BR, d3), row),
        out_shape=jax.ShapeDtypeStruct((N, d3), jnp.float32),
    )(adj, u, b2r, W3)

    out = pl.pallas_call(
        _l3_body,
        grid=grid,
        in_specs=[pl.BlockSpec((_BR, N), row),
                  pl.BlockSpec((N, d3), const),
                  pl.BlockSpec((1, d3), const)],
        out_specs=pl.BlockSpec((_BR, d3), row),
        out_shape=jax.ShapeDtypeStruct((N, d3), jnp.float32),
    )(adj, t, b3r)
    return out
